# baseline (speedup 1.0000x reference)
import jax
import jax.numpy as jnp
from jax import lax
from jax.experimental import pallas as pl
from jax.experimental.pallas import tpu as pltpu
from jax.experimental.pallas import tpu_sc as plsc

N = 200000
D = 100
DP = 128
K = 50000
NPEERS_USED = 3
M = NPEERS_USED * K
NC = 2
NS = 16
L = 16

PER_TILE = 9472
M_PAD = NS * PER_TILE
CH_SC = 6400
CH_TOTAL = NC * CH_SC
PASSES = -(-N // CH_TOTAL)
ROWS_PER_TILE = CH_SC // NS
BND = (N - (PASSES - 1) * CH_TOTAL) % ROWS_PER_TILE
BATCH = 128
G = 3
SEL_CAP = PER_TILE + G * BATCH
DUMP = CH_SC


def _matmul_kernel(x_ref, wt_ref, b_ref, o_ref):
    o_ref[...] = (
        jnp.dot(x_ref[...], wt_ref[...], preferred_element_type=jnp.float32)
        + b_ref[0:1, :]
    )


def _linear(x, wtp, b8p):
    br = 2000
    return pl.pallas_call(
        _matmul_kernel,
        grid=(N // br,),
        in_specs=[
            pl.BlockSpec((br, D), lambda i: (i, 0)),
            pl.BlockSpec((D, DP), lambda i: (0, 0)),
            pl.BlockSpec((8, DP), lambda i: (0, 0)),
        ],
        out_specs=pl.BlockSpec((br, DP), lambda i: (i, 0)),
        out_shape=jax.ShapeDtypeStruct((N, DP), jnp.float32),
    )(x, wtp, b8p)


def _shuffle_body(a_hbm, from_hbm, to_hbm, out_hbm,
                  from_t, to_t, sel_idx, stage_to, stage_off,
                  rows, acc, gsem, ssem, isem):
    c = lax.axis_index("c")
    s = lax.axis_index("s")
    lane = lax.iota(jnp.int32, L)

    base_idx = s * PER_TILE
    pltpu.sync_copy(from_hbm.at[pl.ds(base_idx, PER_TILE)], from_t)
    pltpu.sync_copy(to_hbm.at[pl.ds(base_idx, PER_TILE)], to_t)

    def do_pass(lo, last):
        row0 = lo + s * ROWS_PER_TILE

        if not last:
            pltpu.async_copy(a_hbm.at[pl.ds(row0, ROWS_PER_TILE)],
                             acc.at[pl.ds(s * ROWS_PER_TILE, ROWS_PER_TILE)],
                             isem)
        else:
            @pl.when(row0 + ROWS_PER_TILE <= N)
            def _():
                pltpu.sync_copy(
                    a_hbm.at[pl.ds(row0, ROWS_PER_TILE)],
                    acc.at[pl.ds(s * ROWS_PER_TILE, ROWS_PER_TILE)])

            if BND:
                @pl.when((row0 < N) & (row0 + ROWS_PER_TILE > N))
                def _():
                    pltpu.sync_copy(a_hbm.at[pl.ds(row0, BND)],
                                    acc.at[pl.ds(s * ROWS_PER_TILE, BND)])

        def scan_body(i, ptr):
            idx = i * L + lane
            f = plsc.load_gather(from_t, [idx])
            m = (f >= lo) & (f < lo + CH_SC)
            mi = jnp.where(m, 1, 0).astype(jnp.int32)
            cum = plsc.cumsum(mi)
            pos = ptr + cum - 1
            plsc.store_scatter(sel_idx, [pos], idx, mask=m)
            return ptr + jnp.sum(mi)

        nsel = lax.fori_loop(0, PER_TILE // L, scan_body, jnp.int32(0))

        for q in range(BATCH // L):
            pos = nsel + q * L + lane
            plsc.store_scatter(sel_idx, [pos],
                               jnp.full((L,), PER_TILE, jnp.int32))

        if not last:
            pltpu.make_async_copy(
                a_hbm.at[pl.ds(row0, ROWS_PER_TILE)],
                acc.at[pl.ds(s * ROWS_PER_TILE, ROWS_PER_TILE)],
                isem).wait()
        plsc.subcore_barrier()

        nb = (nsel + BATCH - 1) // BATCH

        def group_body(g, carry):
            for u in range(G):
                jv = g * G + u

                @pl.when(jv < nb)
                def _(u=u, jv=jv):
                    st = stage_to.at[u]
                    so = stage_off.at[u]
                    for q in range(BATCH // L):
                        idx = jv * BATCH + q * L + lane
                        k = plsc.load_gather(sel_idx, [idx])
                        kc = jnp.minimum(k, PER_TILE - 1)
                        t16 = plsc.load_gather(to_t, [kc])
                        f16 = plsc.load_gather(from_t, [kc])
                        off = jnp.where(k >= PER_TILE, DUMP, f16 - lo)
                        st[pl.ds(q * L, L)] = t16
                        so[pl.ds(q * L, L)] = off
                    pltpu.async_copy(a_hbm.at[stage_to.at[u]], rows.at[u],
                                     gsem.at[u])
            for u in range(G):
                jv = g * G + u

                @pl.when(jv < nb)
                def _(u=u):
                    pltpu.make_async_copy(a_hbm.at[stage_to.at[u]],
                                          rows.at[u], gsem.at[u]).wait()
                    pltpu.async_copy(rows.at[u], acc.at[stage_off.at[u]],
                                     ssem.at[u], add=True)
            for u in range(G):
                jv = g * G + u

                @pl.when(jv < nb)
                def _(u=u):
                    pltpu.make_async_copy(rows.at[u],
                                          acc.at[stage_off.at[u]],
                                          ssem.at[u]).wait()
            return carry

        ngrp = (nb + G - 1) // G
        lax.fori_loop(0, ngrp, group_body, jnp.int32(0))
        plsc.subcore_barrier()

        if not last:
            pltpu.sync_copy(acc.at[pl.ds(s * ROWS_PER_TILE, ROWS_PER_TILE)],
                            out_hbm.at[pl.ds(row0, ROWS_PER_TILE)])
        else:
            @pl.when(row0 + ROWS_PER_TILE <= N)
            def _():
                pltpu.sync_copy(
                    acc.at[pl.ds(s * ROWS_PER_TILE, ROWS_PER_TILE)],
                    out_hbm.at[pl.ds(row0, ROWS_PER_TILE)])

            if BND:
                @pl.when((row0 < N) & (row0 + ROWS_PER_TILE > N))
                def _():
                    pltpu.sync_copy(acc.at[pl.ds(s * ROWS_PER_TILE, BND)],
                                    out_hbm.at[pl.ds(row0, BND)])

    def pass_body(p, carry):
        do_pass(p * CH_TOTAL + c * CH_SC, last=False)
        return carry

    lax.fori_loop(0, PASSES - 1, pass_body, jnp.int32(0))
    do_pass((PASSES - 1) * CH_TOTAL + c * CH_SC, last=True)


def _shuffle(a, fp, tp):
    return pl.kernel(
        _shuffle_body,
        out_type=jax.ShapeDtypeStruct((N, DP), jnp.float32),
        mesh=plsc.VectorSubcoreMesh(
            core_axis_name="c", subcore_axis_name="s",
            num_cores=NC, num_subcores=NS),
        scratch_types=[
            pltpu.VMEM((PER_TILE,), jnp.int32),
            pltpu.VMEM((PER_TILE,), jnp.int32),
            pltpu.VMEM((SEL_CAP,), jnp.int32),
            pltpu.VMEM((G, BATCH), jnp.int32),
            pltpu.VMEM((G, BATCH), jnp.int32),
            pltpu.VMEM((G, BATCH, DP), jnp.float32),
            pltpu.VMEM_SHARED((CH_SC + 8, DP), jnp.float32),
            pltpu.SemaphoreType.DMA((G,)),
            pltpu.SemaphoreType.DMA((G,)),
            pltpu.SemaphoreType.DMA,
        ],
        compiler_params=pltpu.CompilerParams(needs_layout_passes=False),
    )(a, fp, tp)


def kernel(input, from_id, to_id, W, b):
    wtp = jnp.zeros((D, DP), jnp.float32).at[:, :D].set(W.T)
    b8p = jnp.zeros((8, DP), jnp.float32).at[:, :D].set(
        jnp.broadcast_to(b, (8, D)))
    a = _linear(input, wtp, b8p)
    pad_f = jnp.full((M_PAD - M,), -1, jnp.int32)
    pad_t = jnp.zeros((M_PAD - M,), jnp.int32)
    fp = jnp.concatenate([from_id[1:].reshape(-1), pad_f])
    tp = jnp.concatenate([to_id[1:].reshape(-1), pad_t])
    out_pad = _shuffle(a, fp, tp)
    return out_pad[:, :D]


# speedup vs baseline: 1.0386x; 1.0386x over previous
import jax
import jax.numpy as jnp
from jax import lax
from jax.experimental import pallas as pl
from jax.experimental.pallas import tpu as pltpu
from jax.experimental.pallas import tpu_sc as plsc

N = 200000
D = 100
DP = 128
K = 50000
NPEERS_USED = 3
M = NPEERS_USED * K
NC = 2
NS = 16
L = 16

PER_TILE = 9472
M_PAD = NS * PER_TILE
CH_SC = 8576
CH_TOTAL = NC * CH_SC
PASSES = -(-N // CH_TOTAL)
ROWS_PER_TILE = CH_SC // NS
BND = (N - (PASSES - 1) * CH_TOTAL) % ROWS_PER_TILE
BATCH = 128
G = 2
SEL_CAP = PER_TILE + G * BATCH
DUMP = CH_SC


def _matmul_kernel(x_ref, wt_ref, b_ref, o_ref):
    o_ref[...] = (
        jnp.dot(x_ref[...], wt_ref[...], preferred_element_type=jnp.float32)
        + b_ref[0:1, :]
    )


def _linear(x, wtp, b8p):
    br = 2000
    return pl.pallas_call(
        _matmul_kernel,
        grid=(N // br,),
        in_specs=[
            pl.BlockSpec((br, D), lambda i: (i, 0)),
            pl.BlockSpec((D, DP), lambda i: (0, 0)),
            pl.BlockSpec((8, DP), lambda i: (0, 0)),
        ],
        out_specs=pl.BlockSpec((br, DP), lambda i: (i, 0)),
        out_shape=jax.ShapeDtypeStruct((N, DP), jnp.float32),
    )(x, wtp, b8p)


def _shuffle_body(a_hbm, from_hbm, to_hbm, out_hbm,
                  from_t, to_t, sel_idx, stage_to, stage_off,
                  rows, acc, gsem, ssem, isem):
    c = lax.axis_index("c")
    s = lax.axis_index("s")
    lane = lax.iota(jnp.int32, L)

    base_idx = s * PER_TILE
    pltpu.sync_copy(from_hbm.at[pl.ds(base_idx, PER_TILE)], from_t)
    pltpu.sync_copy(to_hbm.at[pl.ds(base_idx, PER_TILE)], to_t)

    def do_pass(lo, last):
        row0 = lo + s * ROWS_PER_TILE

        if not last:
            pltpu.async_copy(a_hbm.at[pl.ds(row0, ROWS_PER_TILE)],
                             acc.at[pl.ds(s * ROWS_PER_TILE, ROWS_PER_TILE)],
                             isem)
        else:
            @pl.when(row0 + ROWS_PER_TILE <= N)
            def _():
                pltpu.sync_copy(
                    a_hbm.at[pl.ds(row0, ROWS_PER_TILE)],
                    acc.at[pl.ds(s * ROWS_PER_TILE, ROWS_PER_TILE)])

            @pl.when((row0 < N) & (row0 + ROWS_PER_TILE > N))
            def _():
                pltpu.sync_copy(a_hbm.at[pl.ds(row0, BND)],
                                acc.at[pl.ds(s * ROWS_PER_TILE, BND)])

        def scan_body(i, ptr):
            idx = i * L + lane
            f = plsc.load_gather(from_t, [idx])
            m = (f >= lo) & (f < lo + CH_SC)
            mi = jnp.where(m, 1, 0).astype(jnp.int32)
            cum = plsc.cumsum(mi)
            pos = ptr + cum - 1
            plsc.store_scatter(sel_idx, [pos], idx, mask=m)
            return ptr + jnp.sum(mi)

        nsel = lax.fori_loop(0, PER_TILE // L, scan_body, jnp.int32(0))

        for q in range(BATCH // L):
            pos = nsel + q * L + lane
            plsc.store_scatter(sel_idx, [pos],
                               jnp.full((L,), PER_TILE, jnp.int32))

        if not last:
            pltpu.make_async_copy(
                a_hbm.at[pl.ds(row0, ROWS_PER_TILE)],
                acc.at[pl.ds(s * ROWS_PER_TILE, ROWS_PER_TILE)],
                isem).wait()
        plsc.subcore_barrier()

        nb = (nsel + BATCH - 1) // BATCH

        def group_body(g, carry):
            for u in range(G):
                jv = g * G + u

                @pl.when(jv < nb)
                def _(u=u, jv=jv):
                    st = stage_to.at[u]
                    so = stage_off.at[u]
                    for q in range(BATCH // L):
                        idx = jv * BATCH + q * L + lane
                        k = plsc.load_gather(sel_idx, [idx])
                        kc = jnp.minimum(k, PER_TILE - 1)
                        t16 = plsc.load_gather(to_t, [kc])
                        f16 = plsc.load_gather(from_t, [kc])
                        off = jnp.where(k >= PER_TILE, DUMP, f16 - lo)
                        st[pl.ds(q * L, L)] = t16
                        so[pl.ds(q * L, L)] = off
                    pltpu.async_copy(a_hbm.at[stage_to.at[u]], rows.at[u],
                                     gsem.at[u])
            for u in range(G):
                jv = g * G + u

                @pl.when(jv < nb)
                def _(u=u):
                    pltpu.make_async_copy(a_hbm.at[stage_to.at[u]],
                                          rows.at[u], gsem.at[u]).wait()
                    pltpu.async_copy(rows.at[u], acc.at[stage_off.at[u]],
                                     ssem.at[u], add=True)
            for u in range(G):
                jv = g * G + u

                @pl.when(jv < nb)
                def _(u=u):
                    pltpu.make_async_copy(rows.at[u],
                                          acc.at[stage_off.at[u]],
                                          ssem.at[u]).wait()
            return carry

        ngrp = (nb + G - 1) // G
        lax.fori_loop(0, ngrp, group_body, jnp.int32(0))
        plsc.subcore_barrier()

        if not last:
            pltpu.sync_copy(acc.at[pl.ds(s * ROWS_PER_TILE, ROWS_PER_TILE)],
                            out_hbm.at[pl.ds(row0, ROWS_PER_TILE)])
        else:
            @pl.when(row0 + ROWS_PER_TILE <= N)
            def _():
                pltpu.sync_copy(
                    acc.at[pl.ds(s * ROWS_PER_TILE, ROWS_PER_TILE)],
                    out_hbm.at[pl.ds(row0, ROWS_PER_TILE)])

            @pl.when((row0 < N) & (row0 + ROWS_PER_TILE > N))
            def _():
                pltpu.sync_copy(acc.at[pl.ds(s * ROWS_PER_TILE, BND)],
                                out_hbm.at[pl.ds(row0, BND)])

    def pass_body(p, carry):
        do_pass(p * CH_TOTAL + c * CH_SC, last=False)
        return carry

    lax.fori_loop(0, PASSES - 1, pass_body, jnp.int32(0))
    do_pass((PASSES - 1) * CH_TOTAL + c * CH_SC, last=True)


def _shuffle(a, fp, tp):
    return pl.kernel(
        _shuffle_body,
        out_type=jax.ShapeDtypeStruct((N, DP), jnp.float32),
        mesh=plsc.VectorSubcoreMesh(
            core_axis_name="c", subcore_axis_name="s",
            num_cores=NC, num_subcores=NS),
        scratch_types=[
            pltpu.VMEM((PER_TILE,), jnp.int32),
            pltpu.VMEM((PER_TILE,), jnp.int32),
            pltpu.VMEM((SEL_CAP,), jnp.int32),
            pltpu.VMEM((G, BATCH), jnp.int32),
            pltpu.VMEM((G, BATCH), jnp.int32),
            pltpu.VMEM((G, BATCH, DP), jnp.float32),
            pltpu.VMEM_SHARED((CH_SC + 8, DP), jnp.float32),
            pltpu.SemaphoreType.DMA((G,)),
            pltpu.SemaphoreType.DMA((G,)),
            pltpu.SemaphoreType.DMA,
        ],
        compiler_params=pltpu.CompilerParams(needs_layout_passes=False),
    )(a, fp, tp)


def kernel(input, from_id, to_id, W, b):
    wtp = jnp.zeros((D, DP), jnp.float32).at[:, :D].set(W.T)
    b8p = jnp.zeros((8, DP), jnp.float32).at[:, :D].set(
        jnp.broadcast_to(b, (8, D)))
    a = _linear(input, wtp, b8p)
    pad_f = jnp.full((M_PAD - M,), -1, jnp.int32)
    pad_t = jnp.zeros((M_PAD - M,), jnp.int32)
    fp = jnp.concatenate([from_id[1:].reshape(-1), pad_f])
    tp = jnp.concatenate([to_id[1:].reshape(-1), pad_t])
    out_pad = _shuffle(a, fp, tp)
    return out_pad[:, :D]
